# baseline (speedup 1.0000x reference)
import functools

import jax
import jax.numpy as jnp
from jax import lax
from jax.experimental import pallas as pl
from jax.experimental.pallas import tpu as pltpu
from jax.experimental.pallas import tpu_sc as plsc

N = 10000
D = 128
E = 320000
NN = 10240
NC = 2
NS = 16
NW = NC * NS
CH = 128
J = 81
EPW = J * CH
ET = NW * EPW
ER = E + N
PAD = ET - ER
RPT = NN // NS
DP = 32
NP = D // DP


def _mm1_body(x_ref, w_ref, asv_ref, adv_ref, h4_ref, s_ref, d_ref):
    x = x_ref[...]
    h = lax.dot_general(x, w_ref[...], (((1,), (1,)), ((), ())),
                        preferred_element_type=jnp.float32)
    rows = lax.broadcasted_iota(jnp.int32, (NN, 1), 0)
    h = jnp.where(rows < N, h, 0.0)
    h4_ref[...] = h
    s_ref[...] = jnp.sum(h * asv_ref[...], axis=1, keepdims=True)
    d_ref[...] = jnp.sum(h * adv_ref[...], axis=1, keepdims=True)


def _mm2_body(g_ref, den_ref, b_ref, w_ref, asv_ref, adv_ref,
              h4_ref, s_ref, d_ref):
    y = jnp.maximum(g_ref[...] / (den_ref[...] + 1e-16) + b_ref[...], 0.0)
    h = lax.dot_general(y, w_ref[...], (((1,), (1,)), ((), ())),
                        preferred_element_type=jnp.float32)
    rows = lax.broadcasted_iota(jnp.int32, (NN, 1), 0)
    h = jnp.where(rows < N, h, 0.0)
    h4_ref[...] = h
    s_ref[...] = jnp.sum(h * asv_ref[...], axis=1, keepdims=True)
    d_ref[...] = jnp.sum(h * adv_ref[...], axis=1, keepdims=True)


def _fin_body(g_ref, den_ref, b_ref, keep_ref, o_ref):
    y = jnp.maximum(g_ref[...] / (den_ref[...] + 1e-16) + b_ref[...], 0.0)
    o_ref[...] = y * keep_ref[...]


_MM_OUT = [jax.ShapeDtypeStruct((NN, D), jnp.float32),
           jax.ShapeDtypeStruct((NN, 1), jnp.float32),
           jax.ShapeDtypeStruct((NN, 1), jnp.float32)]


def _mm1(xp, w, asv, adv):
    return pl.pallas_call(_mm1_body, out_shape=_MM_OUT)(xp, w, asv, adv)


def _mm2(agg, den, b, w, asv, adv):
    return pl.pallas_call(_mm2_body, out_shape=_MM_OUT)(agg, den, b, w,
                                                        asv, adv)


def _fin(agg, den, b, keep):
    return pl.pallas_call(
        _fin_body, out_shape=jax.ShapeDtypeStruct((NN, D), jnp.float32),
    )(agg, den, b, keep)


_sc_mesh = plsc.VectorSubcoreMesh(
    core_axis_name="c", subcore_axis_name="s", num_cores=NC, num_subcores=NS)


@functools.partial(
    pl.kernel,
    out_type=[jax.ShapeDtypeStruct((NN, D), jnp.float32),
              jax.ShapeDtypeStruct((NC, NN), jnp.float32)],
    mesh=_sc_mesh,
    compiler_params=pltpu.CompilerParams(needs_layout_passes=False,
                                         use_tc_tiling_on_sc=False),
    scratch_types=[
        pltpu.VMEM((NN,), jnp.float32),
        pltpu.VMEM((NN,), jnp.float32),
        pltpu.VMEM((J, CH), jnp.int32),
        pltpu.VMEM((J, CH), jnp.int32),
        pltpu.VMEM((J, CH), jnp.int32),
        pltpu.VMEM((J, CH), jnp.int32),
        pltpu.VMEM((J, CH), jnp.float32),
        pltpu.VMEM((J, CH), jnp.float32),
        pltpu.VMEM((RPT,), jnp.float32),
        pltpu.VMEM((CH, DP), jnp.float32),
        pltpu.VMEM((CH, DP), jnp.float32),
        pltpu.VMEM((CH, DP), jnp.float32),
        pltpu.VMEM((CH, DP), jnp.float32),
        pltpu.VMEM((CH, DP), jnp.float32),
        pltpu.VMEM((CH, DP), jnp.float32),
        pltpu.VMEM_SHARED((NN,), jnp.float32),
        pltpu.VMEM_SHARED((NN, DP), jnp.float32),
        pltpu.SemaphoreType.DMA,
        pltpu.SemaphoreType.DMA,
        pltpu.SemaphoreType.DMA,
        pltpu.SemaphoreType.DMA,
        pltpu.SemaphoreType.DMA,
        pltpu.SemaphoreType.DMA,
        pltpu.SemaphoreType.DMA,
        pltpu.SemaphoreType.DMA,
        pltpu.SemaphoreType.DMA,
        pltpu.SemaphoreType.DMA,
        pltpu.SemaphoreType.DMA,
        pltpu.SemaphoreType.DMA,
        pltpu.SemaphoreType.DMA,
    ],
)
def _sc_edge(hp_hbm, asrc_hbm, adst_hbm, src_hbm, dst_hbm,
             out_hbm, den_hbm,
             asrc_l, adst_l, src_my, dst_my, src_ot, dst_ot, exb, exb2,
             zvec, rb0, rb1, rb2, rb3, rb4, rb5, den_sh, out_sh,
             sg0, sg1, sg2, sg3, sg4, sg5,
             ss0, ss1, ss2, ss3, ss4, ss5, sem_a):
    c = lax.axis_index("c")
    t = lax.axis_index("s")
    w_my = c * NS + t
    w_ot = (1 - c) * NS + t
    base = t * RPT

    pltpu.sync_copy(asrc_hbm, asrc_l)
    pltpu.sync_copy(adst_hbm, adst_l)
    pltpu.sync_copy(src_hbm.at[w_my], src_my)
    pltpu.sync_copy(dst_hbm.at[w_my], dst_my)
    pltpu.sync_copy(src_hbm.at[w_ot], src_ot)
    pltpu.sync_copy(dst_hbm.at[w_ot], dst_ot)

    zv = jnp.zeros((16,), jnp.float32)

    def _z_row(r, _):
        for q in range(DP // 16):
            rb0[r, pl.ds(q * 16, 16)] = zv
        return 0
    lax.fori_loop(0, CH, _z_row, 0)

    def _z_vec(i, _):
        zvec[pl.ds(i * 16, 16)] = zv
        return 0
    lax.fori_loop(0, RPT // 16, _z_vec, 0)

    pltpu.sync_copy(zvec, den_sh.at[pl.ds(base, RPT)])
    plsc.subcore_barrier()

    def _pass_a(sref, dref, eref):
        def _pa(j, _):
            for k in range(CH // 16):
                sl = pl.ds(k * 16, 16)
                e = (plsc.load_gather(asrc_l, [sref[j, sl]]) +
                     plsc.load_gather(adst_l, [dref[j, sl]]))
                e = jnp.where(e >= 0.0, e, e * 0.2)
                eref[j, sl] = jnp.exp(e)
            pltpu.async_copy(eref.at[j], den_sh.at[dref.at[j]], sem_a,
                             add=True)
            return 0
        lax.fori_loop(0, J, _pa, 0)

    _pass_a(src_my, dst_my, exb)
    _pass_a(src_ot, dst_ot, exb2)

    def _drain_a(i, _):
        pltpu.make_async_copy(exb.at[0], den_sh.at[dst_my.at[0]],
                              sem_a).wait()
        return 0
    lax.fori_loop(0, 2 * J, _drain_a, 0)


    for kp in range(NP // NC):
        p = 2 * kp + c

        lax.fori_loop(0, CH, _z_row, 0)
        for i in range(RPT // CH):
            pltpu.sync_copy(rb0, out_sh.at[pl.ds(base + i * CH, CH)])
        plsc.subcore_barrier()
        if kp == 0:
            pltpu.sync_copy(den_sh.at[pl.ds(base, RPT)],
                            den_hbm.at[c].at[pl.ds(base, RPT)])

        def _rows(sref, dref, eref):
            rbs = (rb0, rb1, rb2, rb3, rb4, rb5)
            sgs = (sg0, sg1, sg2, sg3, sg4, sg5)
            sss = (ss0, ss1, ss2, ss3, ss4, ss5)
            LA = 4

            def _start_g(j, b):
                pltpu.async_copy(hp_hbm.at[p].at[sref.at[j]], rbs[b], sgs[b])

            def _wait_g(b):
                pltpu.make_async_copy(hp_hbm.at[p].at[sref.at[0]], rbs[b],
                                      sgs[b]).wait()

            def _wait_s(b):
                pltpu.make_async_copy(rbs[b], out_sh.at[dref.at[0]],
                                      sss[b]).wait()

            def _scale(j, b):
                rb = rbs[b]

                def _sc16(g, _):
                    avec = eref[j, pl.ds(g * 16, 16)]
                    for r in range(16):
                        a = avec[r]
                        row = g * 16 + r
                        for q in range(DP // 16):
                            sl = pl.ds(q * 16, 16)
                            rb[row, sl] = rb[row, sl] * a
                    return 0
                lax.fori_loop(0, CH // 16, _sc16, 0)

            for b in range(LA):
                _start_g(b, b)

            def _body(j2, _):
                for u in range(6):
                    j = 6 * j2 + u
                    bn = (u + LA) % 6

                    @pl.when(j < J)
                    def _():
                        _wait_g(u)
                        _scale(j, u)
                        pltpu.async_copy(rbs[u], out_sh.at[dref.at[j]],
                                         sss[u], add=True)

                        @pl.when(j >= 2)
                        def _():
                            _wait_s(bn)

                        @pl.when(j + LA < J)
                        def _():
                            _start_g(j + LA, bn)
                return 0
            lax.fori_loop(0, (J + 5) // 6, _body, 0)
            _wait_s((J - 2) % 6)
            _wait_s((J - 1) % 6)

        _rows(src_my, dst_my, exb)
        _rows(src_ot, dst_ot, exb2)
        plsc.subcore_barrier()

        pltpu.sync_copy(out_sh.at[pl.ds(base, RPT)],
                        out_hbm.at[pl.ds(base, RPT), pl.ds(p * DP, DP)])
        plsc.subcore_barrier()


def kernel(x, edge_index, batch_size, framework,
           W1, a_src1, a_dst1, b1,
           W2, a_src2, a_dst2, b2,
           W3, a_src3, a_dst3, b3):
    f32 = jnp.float32
    xp = jnp.zeros((NN, D), f32).at[:N].set(x)
    loop = jnp.arange(N, dtype=jnp.int32)
    src = jnp.concatenate(
        [edge_index[0], loop, jnp.zeros((PAD,), jnp.int32)]).reshape(NW, J, CH)
    dst = jnp.concatenate(
        [edge_index[1], loop, jnp.full((PAD,), N, jnp.int32)]).reshape(NW, J, CH)

    def edge_phase(h, s, d):
        h4 = jnp.transpose(h.reshape(NN, NP, DP), (1, 0, 2))
        agg, den = _sc_edge(h4, s.reshape(NN), d.reshape(NN), src, dst)
        return agg, den[0].reshape(NN, 1)

    h4, s, d = _mm1(xp, W1, a_src1.reshape(1, D), a_dst1.reshape(1, D))
    agg, den = edge_phase(h4, s, d)
    h4, s, d = _mm2(agg, den, b1.reshape(1, D), W2,
                    a_src2.reshape(1, D), a_dst2.reshape(1, D))
    agg, den = edge_phase(h4, s, d)
    h4, s, d = _mm2(agg, den, b2.reshape(1, D), W3,
                    a_src3.reshape(1, D), a_dst3.reshape(1, D))
    agg, den = edge_phase(h4, s, d)

    limit = jnp.where(framework != 0, jnp.asarray(N, dtype=jnp.int32),
                      batch_size)
    keep = (jnp.arange(NN, dtype=jnp.int32) < limit).astype(f32).reshape(NN, 1)
    out = _fin(agg, den, b3.reshape(1, D), keep)
    return out[:N]


# speedup vs baseline: 1.0852x; 1.0852x over previous
import functools

import jax
import jax.numpy as jnp
from jax import lax
from jax.experimental import pallas as pl
from jax.experimental.pallas import tpu as pltpu
from jax.experimental.pallas import tpu_sc as plsc

N = 10000
D = 128
E = 320000
NN = 10240
NC = 2
NS = 16
NW = NC * NS
CH = 128
J = 81
EPW = J * CH
ET = NW * EPW
ER = E + N
PAD = ET - ER
RPT = NN // NS
DP = 32
NP = D // DP


def _split_store(h, h4_ref):
    for p in range(NP):
        h4_ref[p, :, :] = h[:, p * DP:(p + 1) * DP]


def _mm1_body(x_ref, w_ref, asv_ref, adv_ref, h4_ref, s_ref, d_ref):
    x = x_ref[...]
    h = lax.dot_general(x, w_ref[...], (((1,), (1,)), ((), ())),
                        preferred_element_type=jnp.float32)
    rows = lax.broadcasted_iota(jnp.int32, (NN, 1), 0)
    h = jnp.where(rows < N, h, 0.0)
    _split_store(h, h4_ref)
    s_ref[...] = jnp.sum(h * asv_ref[...], axis=1, keepdims=True)
    d_ref[...] = jnp.sum(h * adv_ref[...], axis=1, keepdims=True)


def _mm2_body(g_ref, den_ref, b_ref, w_ref, asv_ref, adv_ref,
              h4_ref, s_ref, d_ref):
    y = jnp.maximum(g_ref[...] / (den_ref[...] + 1e-16) + b_ref[...], 0.0)
    h = lax.dot_general(y, w_ref[...], (((1,), (1,)), ((), ())),
                        preferred_element_type=jnp.float32)
    rows = lax.broadcasted_iota(jnp.int32, (NN, 1), 0)
    h = jnp.where(rows < N, h, 0.0)
    _split_store(h, h4_ref)
    s_ref[...] = jnp.sum(h * asv_ref[...], axis=1, keepdims=True)
    d_ref[...] = jnp.sum(h * adv_ref[...], axis=1, keepdims=True)


def _fin_body(g_ref, den_ref, b_ref, keep_ref, o_ref):
    y = jnp.maximum(g_ref[...] / (den_ref[...] + 1e-16) + b_ref[...], 0.0)
    o_ref[...] = y * keep_ref[...]


_MM_OUT = [jax.ShapeDtypeStruct((NP, NN, DP), jnp.float32),
           jax.ShapeDtypeStruct((NN, 1), jnp.float32),
           jax.ShapeDtypeStruct((NN, 1), jnp.float32)]


def _mm1(xp, w, asv, adv):
    return pl.pallas_call(_mm1_body, out_shape=_MM_OUT)(xp, w, asv, adv)


def _mm2(agg, den, b, w, asv, adv):
    return pl.pallas_call(_mm2_body, out_shape=_MM_OUT)(agg, den, b, w,
                                                        asv, adv)


def _fin(agg, den, b, keep):
    return pl.pallas_call(
        _fin_body, out_shape=jax.ShapeDtypeStruct((NN, D), jnp.float32),
    )(agg, den, b, keep)


_sc_mesh = plsc.VectorSubcoreMesh(
    core_axis_name="c", subcore_axis_name="s", num_cores=NC, num_subcores=NS)


@functools.partial(
    pl.kernel,
    out_type=[jax.ShapeDtypeStruct((NN, D), jnp.float32),
              jax.ShapeDtypeStruct((NC, NN), jnp.float32)],
    mesh=_sc_mesh,
    compiler_params=pltpu.CompilerParams(needs_layout_passes=False,
                                         use_tc_tiling_on_sc=False),
    scratch_types=[
        pltpu.VMEM((NN,), jnp.float32),
        pltpu.VMEM((NN,), jnp.float32),
        pltpu.VMEM((J, CH), jnp.int32),
        pltpu.VMEM((J, CH), jnp.int32),
        pltpu.VMEM((J, CH), jnp.int32),
        pltpu.VMEM((J, CH), jnp.int32),
        pltpu.VMEM((J, CH), jnp.float32),
        pltpu.VMEM((J, CH), jnp.float32),
        pltpu.VMEM((RPT,), jnp.float32),
        pltpu.VMEM((CH, DP), jnp.float32),
        pltpu.VMEM((CH, DP), jnp.float32),
        pltpu.VMEM((CH, DP), jnp.float32),
        pltpu.VMEM((CH, DP), jnp.float32),
        pltpu.VMEM((CH, DP), jnp.float32),
        pltpu.VMEM((CH, DP), jnp.float32),
        pltpu.VMEM_SHARED((NN,), jnp.float32),
        pltpu.VMEM_SHARED((NN, DP), jnp.float32),
        pltpu.SemaphoreType.DMA,
        pltpu.SemaphoreType.DMA,
        pltpu.SemaphoreType.DMA,
        pltpu.SemaphoreType.DMA,
        pltpu.SemaphoreType.DMA,
        pltpu.SemaphoreType.DMA,
        pltpu.SemaphoreType.DMA,
        pltpu.SemaphoreType.DMA,
        pltpu.SemaphoreType.DMA,
        pltpu.SemaphoreType.DMA,
        pltpu.SemaphoreType.DMA,
        pltpu.SemaphoreType.DMA,
        pltpu.SemaphoreType.DMA,
    ],
)
def _sc_edge(hp_hbm, asrc_hbm, adst_hbm, src_hbm, dst_hbm,
             out_hbm, den_hbm,
             asrc_l, adst_l, src_my, dst_my, src_ot, dst_ot, exb, exb2,
             zvec, rb0, rb1, rb2, rb3, rb4, rb5, den_sh, out_sh,
             sg0, sg1, sg2, sg3, sg4, sg5,
             ss0, ss1, ss2, ss3, ss4, ss5, sem_a):
    c = lax.axis_index("c")
    t = lax.axis_index("s")
    w_my = c * NS + t
    w_ot = (1 - c) * NS + t
    base = t * RPT

    pltpu.sync_copy(asrc_hbm, asrc_l)
    pltpu.sync_copy(adst_hbm, adst_l)
    pltpu.sync_copy(src_hbm.at[w_my], src_my)
    pltpu.sync_copy(dst_hbm.at[w_my], dst_my)
    pltpu.sync_copy(src_hbm.at[w_ot], src_ot)
    pltpu.sync_copy(dst_hbm.at[w_ot], dst_ot)

    zv = jnp.zeros((16,), jnp.float32)

    def _z_row(r, _):
        for q in range(DP // 16):
            rb0[r, pl.ds(q * 16, 16)] = zv
        return 0
    lax.fori_loop(0, CH, _z_row, 0)

    def _z_vec(i, _):
        zvec[pl.ds(i * 16, 16)] = zv
        return 0
    lax.fori_loop(0, RPT // 16, _z_vec, 0)

    pltpu.sync_copy(zvec, den_sh.at[pl.ds(base, RPT)])
    plsc.subcore_barrier()


    for kp in range(NP // NC):
        p = 2 * kp + c

        lax.fori_loop(0, CH, _z_row, 0)
        for i in range(RPT // CH):
            pltpu.sync_copy(rb0, out_sh.at[pl.ds(base + i * CH, CH)])
        plsc.subcore_barrier()
        if kp == 1:
            pltpu.sync_copy(den_sh.at[pl.ds(base, RPT)],
                            den_hbm.at[c].at[pl.ds(base, RPT)])

        def _rows(sref, dref, eref, compute_ex):
            rbs = (rb0, rb1, rb2, rb3, rb4, rb5)
            sgs = (sg0, sg1, sg2, sg3, sg4, sg5)
            sss = (ss0, ss1, ss2, ss3, ss4, ss5)
            LA = 4

            def _start_g(j, b):
                pltpu.async_copy(hp_hbm.at[p].at[sref.at[j]], rbs[b], sgs[b])

            def _wait_g(b):
                pltpu.make_async_copy(hp_hbm.at[p].at[sref.at[0]], rbs[b],
                                      sgs[b]).wait()

            def _wait_s(b):
                pltpu.make_async_copy(rbs[b], out_sh.at[dref.at[0]],
                                      sss[b]).wait()

            def _scale(j, b):
                rb = rbs[b]

                def _sc16(g, _):
                    sl16 = pl.ds(g * 16, 16)
                    if compute_ex:
                        e = (plsc.load_gather(asrc_l, [sref[j, sl16]]) +
                             plsc.load_gather(adst_l, [dref[j, sl16]]))
                        e = jnp.where(e >= 0.0, e, e * 0.2)
                        avec = jnp.exp(e)
                        eref[j, sl16] = avec
                    else:
                        avec = eref[j, sl16]
                    for r in range(16):
                        a = avec[r]
                        row = g * 16 + r
                        for q in range(DP // 16):
                            sl = pl.ds(q * 16, 16)
                            rb[row, sl] = rb[row, sl] * a
                    return 0
                lax.fori_loop(0, CH // 16, _sc16, 0)

            for b in range(LA):
                _start_g(b, b)

            def _body(j2, _):
                for u in range(6):
                    j = 6 * j2 + u
                    bn = (u + LA) % 6

                    @pl.when(j < J)
                    def _():
                        _wait_g(u)
                        _scale(j, u)
                        pltpu.async_copy(rbs[u], out_sh.at[dref.at[j]],
                                         sss[u], add=True)
                        if compute_ex:
                            pltpu.async_copy(eref.at[j],
                                             den_sh.at[dref.at[j]],
                                             sem_a, add=True)

                        @pl.when(j >= 2)
                        def _():
                            _wait_s(bn)

                        @pl.when(j + LA < J)
                        def _():
                            _start_g(j + LA, bn)
                return 0
            lax.fori_loop(0, (J + 5) // 6, _body, 0)
            _wait_s((J - 2) % 6)
            _wait_s((J - 1) % 6)

        _rows(src_my, dst_my, exb, kp == 0)
        _rows(src_ot, dst_ot, exb2, kp == 0)
        if kp == 0:
            def _drain_a(i, _):
                pltpu.make_async_copy(exb.at[0], den_sh.at[dst_my.at[0]],
                                      sem_a).wait()
                return 0
            lax.fori_loop(0, 2 * J, _drain_a, 0)
        plsc.subcore_barrier()

        pltpu.sync_copy(out_sh.at[pl.ds(base, RPT)],
                        out_hbm.at[pl.ds(base, RPT), pl.ds(p * DP, DP)])
        plsc.subcore_barrier()


def kernel(x, edge_index, batch_size, framework,
           W1, a_src1, a_dst1, b1,
           W2, a_src2, a_dst2, b2,
           W3, a_src3, a_dst3, b3):
    f32 = jnp.float32
    xp = jnp.zeros((NN, D), f32).at[:N].set(x)
    loop = jnp.arange(N, dtype=jnp.int32)
    src = jnp.concatenate(
        [edge_index[0], loop, jnp.zeros((PAD,), jnp.int32)]).reshape(NW, J, CH)
    dst = jnp.concatenate(
        [edge_index[1], loop, jnp.full((PAD,), N, jnp.int32)]).reshape(NW, J, CH)

    def edge_phase(h4, s, d):
        agg, den = _sc_edge(h4, s.reshape(NN), d.reshape(NN), src, dst)
        return agg, den[0].reshape(NN, 1)

    h4, s, d = _mm1(xp, W1, a_src1.reshape(1, D), a_dst1.reshape(1, D))
    agg, den = edge_phase(h4, s, d)
    h4, s, d = _mm2(agg, den, b1.reshape(1, D), W2,
                    a_src2.reshape(1, D), a_dst2.reshape(1, D))
    agg, den = edge_phase(h4, s, d)
    h4, s, d = _mm2(agg, den, b2.reshape(1, D), W3,
                    a_src3.reshape(1, D), a_dst3.reshape(1, D))
    agg, den = edge_phase(h4, s, d)

    limit = jnp.where(framework != 0, jnp.asarray(N, dtype=jnp.int32),
                      batch_size)
    keep = (jnp.arange(NN, dtype=jnp.int32) < limit).astype(f32).reshape(NN, 1)
    out = _fin(agg, den, b3.reshape(1, D), keep)
    return out[:N]


# speedup vs baseline: 1.1014x; 1.0149x over previous
import functools

import jax
import jax.numpy as jnp
from jax import lax
from jax.experimental import pallas as pl
from jax.experimental.pallas import tpu as pltpu
from jax.experimental.pallas import tpu_sc as plsc

N = 10000
D = 128
E = 320000
NN = 10240
NC = 2
NS = 16
NW = NC * NS
CH = 128
J = 81
EPW = J * CH
ET = NW * EPW
ER = E + N
PAD = ET - ER
RPT = NN // NS
DP = 32
NP = D // DP


def _split_store(h, h4_ref):
    for p in range(NP):
        h4_ref[p, :, :] = h[:, p * DP:(p + 1) * DP]


def _mm1_body(x_ref, w_ref, asv_ref, adv_ref, h4_ref, s_ref, d_ref):
    x = x_ref[...]
    h = lax.dot_general(x, w_ref[...], (((1,), (1,)), ((), ())),
                        preferred_element_type=jnp.float32)
    rows = lax.broadcasted_iota(jnp.int32, (NN, 1), 0)
    h = jnp.where(rows < N, h, 0.0)
    _split_store(h, h4_ref)
    s_ref[...] = jnp.sum(h * asv_ref[...], axis=1, keepdims=True)
    d_ref[...] = jnp.sum(h * adv_ref[...], axis=1, keepdims=True)


def _mm2_body(g_ref, den_ref, b_ref, w_ref, asv_ref, adv_ref,
              h4_ref, s_ref, d_ref):
    y = jnp.maximum(g_ref[...] / (den_ref[...] + 1e-16) + b_ref[...], 0.0)
    h = lax.dot_general(y, w_ref[...], (((1,), (1,)), ((), ())),
                        preferred_element_type=jnp.float32)
    rows = lax.broadcasted_iota(jnp.int32, (NN, 1), 0)
    h = jnp.where(rows < N, h, 0.0)
    _split_store(h, h4_ref)
    s_ref[...] = jnp.sum(h * asv_ref[...], axis=1, keepdims=True)
    d_ref[...] = jnp.sum(h * adv_ref[...], axis=1, keepdims=True)


def _fin_body(g_ref, den_ref, b_ref, keep_ref, o_ref):
    y = jnp.maximum(g_ref[...] / (den_ref[...] + 1e-16) + b_ref[...], 0.0)
    o_ref[...] = y * keep_ref[...]


_MM_OUT = [jax.ShapeDtypeStruct((NP, NN, DP), jnp.float32),
           jax.ShapeDtypeStruct((NN, 1), jnp.float32),
           jax.ShapeDtypeStruct((NN, 1), jnp.float32)]


def _mm1(xp, w, asv, adv):
    return pl.pallas_call(_mm1_body, out_shape=_MM_OUT)(xp, w, asv, adv)


def _mm2(agg, den, b, w, asv, adv):
    return pl.pallas_call(_mm2_body, out_shape=_MM_OUT)(agg, den, b, w,
                                                        asv, adv)


def _fin(agg, den, b, keep):
    return pl.pallas_call(
        _fin_body, out_shape=jax.ShapeDtypeStruct((NN, D), jnp.float32),
    )(agg, den, b, keep)


_sc_mesh = plsc.VectorSubcoreMesh(
    core_axis_name="c", subcore_axis_name="s", num_cores=NC, num_subcores=NS)


@functools.partial(
    pl.kernel,
    out_type=[jax.ShapeDtypeStruct((NN, D), jnp.float32),
              jax.ShapeDtypeStruct((NC, NN), jnp.float32)],
    mesh=_sc_mesh,
    compiler_params=pltpu.CompilerParams(needs_layout_passes=False,
                                         use_tc_tiling_on_sc=False),
    scratch_types=[
        pltpu.VMEM((NN,), jnp.float32),
        pltpu.VMEM((NN,), jnp.float32),
        pltpu.VMEM((J, CH), jnp.int32),
        pltpu.VMEM((J, CH), jnp.int32),
        pltpu.VMEM((J, CH), jnp.int32),
        pltpu.VMEM((J, CH), jnp.int32),
        pltpu.VMEM((J, CH), jnp.float32),
        pltpu.VMEM((J, CH), jnp.float32),
        pltpu.VMEM((RPT,), jnp.float32),
        pltpu.VMEM((CH, DP), jnp.float32),
        pltpu.VMEM((CH, DP), jnp.float32),
        pltpu.VMEM((CH, DP), jnp.float32),
        pltpu.VMEM((CH, DP), jnp.float32),
        pltpu.VMEM((CH, DP), jnp.float32),
        pltpu.VMEM((CH, DP), jnp.float32),
        pltpu.VMEM_SHARED((NN,), jnp.float32),
        pltpu.VMEM_SHARED((NN, DP), jnp.float32),
        pltpu.SemaphoreType.DMA,
        pltpu.SemaphoreType.DMA,
        pltpu.SemaphoreType.DMA,
        pltpu.SemaphoreType.DMA,
        pltpu.SemaphoreType.DMA,
        pltpu.SemaphoreType.DMA,
        pltpu.SemaphoreType.DMA,
        pltpu.SemaphoreType.DMA,
        pltpu.SemaphoreType.DMA,
        pltpu.SemaphoreType.DMA,
        pltpu.SemaphoreType.DMA,
        pltpu.SemaphoreType.DMA,
        pltpu.SemaphoreType.DMA,
    ],
)
def _sc_edge(hp_hbm, asrc_hbm, adst_hbm, src_hbm, dst_hbm,
             out_hbm, den_hbm,
             asrc_l, adst_l, src_my, dst_my, src_ot, dst_ot, exb, exb2,
             zvec, rb0, rb1, rb2, rb3, rb4, rb5, den_sh, out_sh,
             sg0, sg1, sg2, sg3, sg4, sg5,
             ss0, ss1, ss2, ss3, ss4, ss5, sem_a):
    c = lax.axis_index("c")
    t = lax.axis_index("s")
    w_my = c * NS + t
    w_ot = (1 - c) * NS + t
    base = t * RPT

    pltpu.async_copy(asrc_hbm, asrc_l, sg0)
    pltpu.async_copy(adst_hbm, adst_l, sg1)
    pltpu.async_copy(src_hbm.at[w_my], src_my, sg2)
    pltpu.async_copy(dst_hbm.at[w_my], dst_my, sg3)
    pltpu.async_copy(src_hbm.at[w_ot], src_ot, sg4)
    pltpu.async_copy(dst_hbm.at[w_ot], dst_ot, sg5)

    zv = jnp.zeros((16,), jnp.float32)

    def _z_row(r, _):
        for q in range(DP // 16):
            rb0[r, pl.ds(q * 16, 16)] = zv
        return 0
    lax.fori_loop(0, CH, _z_row, 0)

    def _z_vec(i, _):
        zvec[pl.ds(i * 16, 16)] = zv
        return 0
    lax.fori_loop(0, RPT // 16, _z_vec, 0)

    pltpu.sync_copy(zvec, den_sh.at[pl.ds(base, RPT)])
    pltpu.make_async_copy(asrc_hbm, asrc_l, sg0).wait()
    pltpu.make_async_copy(adst_hbm, adst_l, sg1).wait()
    pltpu.make_async_copy(src_hbm.at[w_my], src_my, sg2).wait()
    pltpu.make_async_copy(dst_hbm.at[w_my], dst_my, sg3).wait()
    pltpu.make_async_copy(src_hbm.at[w_ot], src_ot, sg4).wait()
    pltpu.make_async_copy(dst_hbm.at[w_ot], dst_ot, sg5).wait()
    plsc.subcore_barrier()


    for kp in range(NP // NC):
        p = 2 * kp + c

        lax.fori_loop(0, CH, _z_row, 0)
        for i in range(RPT // CH):
            pltpu.sync_copy(rb0, out_sh.at[pl.ds(base + i * CH, CH)])
        plsc.subcore_barrier()
        if kp == 1:
            pltpu.sync_copy(den_sh.at[pl.ds(base, RPT)],
                            den_hbm.at[c].at[pl.ds(base, RPT)])

        def _rows(sref, dref, eref, compute_ex):
            rbs = (rb0, rb1, rb2, rb3, rb4, rb5)
            sgs = (sg0, sg1, sg2, sg3, sg4, sg5)
            sss = (ss0, ss1, ss2, ss3, ss4, ss5)
            LA = 4

            def _start_g(j, b):
                pltpu.async_copy(hp_hbm.at[p].at[sref.at[j]], rbs[b], sgs[b])

            def _wait_g(b):
                pltpu.make_async_copy(hp_hbm.at[p].at[sref.at[0]], rbs[b],
                                      sgs[b]).wait()

            def _wait_s(b):
                pltpu.make_async_copy(rbs[b], out_sh.at[dref.at[0]],
                                      sss[b]).wait()

            def _scale(j, b):
                rb = rbs[b]

                def _sc16(g, _):
                    sl16 = pl.ds(g * 16, 16)
                    if compute_ex:
                        e = (plsc.load_gather(asrc_l, [sref[j, sl16]]) +
                             plsc.load_gather(adst_l, [dref[j, sl16]]))
                        e = jnp.where(e >= 0.0, e, e * 0.2)
                        avec = jnp.exp(e)
                        eref[j, sl16] = avec
                    else:
                        avec = eref[j, sl16]
                    for r in range(16):
                        a = avec[r]
                        row = g * 16 + r
                        for q in range(DP // 16):
                            sl = pl.ds(q * 16, 16)
                            rb[row, sl] = rb[row, sl] * a
                    return 0
                lax.fori_loop(0, CH // 16, _sc16, 0)

            for b in range(LA):
                _start_g(b, b)

            def _body(j2, _):
                for u in range(6):
                    j = 6 * j2 + u
                    bn = (u + LA) % 6

                    @pl.when(j < J)
                    def _():
                        _wait_g(u)
                        _scale(j, u)
                        pltpu.async_copy(rbs[u], out_sh.at[dref.at[j]],
                                         sss[u], add=True)
                        if compute_ex:
                            pltpu.async_copy(eref.at[j],
                                             den_sh.at[dref.at[j]],
                                             sem_a, add=True)

                        @pl.when(j >= 2)
                        def _():
                            _wait_s(bn)

                        @pl.when(j + LA < J)
                        def _():
                            _start_g(j + LA, bn)
                return 0
            lax.fori_loop(0, (J + 5) // 6, _body, 0)
            _wait_s((J - 2) % 6)
            _wait_s((J - 1) % 6)

        _rows(src_my, dst_my, exb, kp == 0)
        _rows(src_ot, dst_ot, exb2, kp == 0)
        if kp == 0:
            def _drain_a(i, _):
                pltpu.make_async_copy(exb.at[0], den_sh.at[dst_my.at[0]],
                                      sem_a).wait()
                return 0
            lax.fori_loop(0, 2 * J, _drain_a, 0)
        plsc.subcore_barrier()

        pltpu.sync_copy(out_sh.at[pl.ds(base, RPT)],
                        out_hbm.at[pl.ds(base, RPT), pl.ds(p * DP, DP)])
        plsc.subcore_barrier()


def kernel(x, edge_index, batch_size, framework,
           W1, a_src1, a_dst1, b1,
           W2, a_src2, a_dst2, b2,
           W3, a_src3, a_dst3, b3):
    f32 = jnp.float32
    xp = jnp.zeros((NN, D), f32).at[:N].set(x)
    loop = jnp.arange(N, dtype=jnp.int32)
    src = jnp.concatenate(
        [edge_index[0], loop, jnp.zeros((PAD,), jnp.int32)]).reshape(NW, J, CH)
    dst = jnp.concatenate(
        [edge_index[1], loop, jnp.full((PAD,), N, jnp.int32)]).reshape(NW, J, CH)

    def edge_phase(h4, s, d):
        agg, den = _sc_edge(h4, s.reshape(NN), d.reshape(NN), src, dst)
        return agg, den[0].reshape(NN, 1)

    h4, s, d = _mm1(xp, W1, a_src1.reshape(1, D), a_dst1.reshape(1, D))
    agg, den = edge_phase(h4, s, d)
    h4, s, d = _mm2(agg, den, b1.reshape(1, D), W2,
                    a_src2.reshape(1, D), a_dst2.reshape(1, D))
    agg, den = edge_phase(h4, s, d)
    h4, s, d = _mm2(agg, den, b2.reshape(1, D), W3,
                    a_src3.reshape(1, D), a_dst3.reshape(1, D))
    agg, den = edge_phase(h4, s, d)

    limit = jnp.where(framework != 0, jnp.asarray(N, dtype=jnp.int32),
                      batch_size)
    keep = (jnp.arange(NN, dtype=jnp.int32) < limit).astype(f32).reshape(NN, 1)
    out = _fin(agg, den, b3.reshape(1, D), keep)
    return out[:N]


# speedup vs baseline: 1.1123x; 1.0099x over previous
import functools

import jax
import jax.numpy as jnp
from jax import lax
from jax.experimental import pallas as pl
from jax.experimental.pallas import tpu as pltpu
from jax.experimental.pallas import tpu_sc as plsc

N = 10000
D = 128
E = 320000
NN = 10240
NC = 2
NS = 16
NW = NC * NS
CH = 128
J = 81
EPW = J * CH
ET = NW * EPW
ER = E + N
PAD = ET - ER
RPT = NN // NS
DP = 32
NP = D // DP


def _split_store(h, h4_ref):
    for p in range(NP):
        h4_ref[p, :, :] = h[:, p * DP:(p + 1) * DP]


def _mm1_body(x_ref, w_ref, asv_ref, adv_ref, h4_ref, s_ref, d_ref):
    x = x_ref[...]
    h = lax.dot_general(x, w_ref[...], (((1,), (1,)), ((), ())),
                        preferred_element_type=jnp.float32)
    h = jnp.concatenate([h, jnp.zeros((NN - N, D), jnp.float32)], axis=0)
    _split_store(h, h4_ref)
    s_ref[...] = jnp.sum(h * asv_ref[...], axis=1, keepdims=True)
    d_ref[...] = jnp.sum(h * adv_ref[...], axis=1, keepdims=True)


def _mm2_body(g_ref, den_ref, b_ref, w_ref, asv_ref, adv_ref,
              h4_ref, s_ref, d_ref):
    y = jnp.maximum(g_ref[...] / (den_ref[...] + 1e-16) + b_ref[...], 0.0)
    h = lax.dot_general(y, w_ref[...], (((1,), (1,)), ((), ())),
                        preferred_element_type=jnp.float32)
    rows = lax.broadcasted_iota(jnp.int32, (NN, 1), 0)
    h = jnp.where(rows < N, h, 0.0)
    _split_store(h, h4_ref)
    s_ref[...] = jnp.sum(h * asv_ref[...], axis=1, keepdims=True)
    d_ref[...] = jnp.sum(h * adv_ref[...], axis=1, keepdims=True)


def _fin_body(g_ref, den_ref, b_ref, keep_ref, o_ref):
    y = jnp.maximum(g_ref[...] / (den_ref[...] + 1e-16) + b_ref[...], 0.0)
    o_ref[...] = (y * keep_ref[...])[:N, :]


_MM_OUT = [jax.ShapeDtypeStruct((NP, NN, DP), jnp.float32),
           jax.ShapeDtypeStruct((NN, 1), jnp.float32),
           jax.ShapeDtypeStruct((NN, 1), jnp.float32)]


def _mm1(xp, w, asv, adv):
    return pl.pallas_call(_mm1_body, out_shape=_MM_OUT)(xp, w, asv, adv)


def _mm2(agg, den, b, w, asv, adv):
    return pl.pallas_call(_mm2_body, out_shape=_MM_OUT)(agg, den, b, w,
                                                        asv, adv)


def _fin(agg, den, b, keep):
    return pl.pallas_call(
        _fin_body, out_shape=jax.ShapeDtypeStruct((N, D), jnp.float32),
    )(agg, den, b, keep)


_sc_mesh = plsc.VectorSubcoreMesh(
    core_axis_name="c", subcore_axis_name="s", num_cores=NC, num_subcores=NS)


@functools.partial(
    pl.kernel,
    out_type=[jax.ShapeDtypeStruct((NN, D), jnp.float32),
              jax.ShapeDtypeStruct((NC, NN), jnp.float32)],
    mesh=_sc_mesh,
    compiler_params=pltpu.CompilerParams(needs_layout_passes=False,
                                         use_tc_tiling_on_sc=False),
    scratch_types=[
        pltpu.VMEM((NN,), jnp.float32),
        pltpu.VMEM((NN,), jnp.float32),
        pltpu.VMEM((J, CH), jnp.int32),
        pltpu.VMEM((J, CH), jnp.int32),
        pltpu.VMEM((J, CH), jnp.int32),
        pltpu.VMEM((J, CH), jnp.int32),
        pltpu.VMEM((J, CH), jnp.float32),
        pltpu.VMEM((J, CH), jnp.float32),
        pltpu.VMEM((RPT,), jnp.float32),
        pltpu.VMEM((CH, DP), jnp.float32),
        pltpu.VMEM((CH, DP), jnp.float32),
        pltpu.VMEM((CH, DP), jnp.float32),
        pltpu.VMEM((CH, DP), jnp.float32),
        pltpu.VMEM((CH, DP), jnp.float32),
        pltpu.VMEM((CH, DP), jnp.float32),
        pltpu.VMEM_SHARED((NN,), jnp.float32),
        pltpu.VMEM_SHARED((NN, DP), jnp.float32),
        pltpu.SemaphoreType.DMA,
        pltpu.SemaphoreType.DMA,
        pltpu.SemaphoreType.DMA,
        pltpu.SemaphoreType.DMA,
        pltpu.SemaphoreType.DMA,
        pltpu.SemaphoreType.DMA,
        pltpu.SemaphoreType.DMA,
        pltpu.SemaphoreType.DMA,
        pltpu.SemaphoreType.DMA,
        pltpu.SemaphoreType.DMA,
        pltpu.SemaphoreType.DMA,
        pltpu.SemaphoreType.DMA,
        pltpu.SemaphoreType.DMA,
    ],
)
def _sc_edge(hp_hbm, asrc_hbm, adst_hbm, src_hbm, dst_hbm,
             out_hbm, den_hbm,
             asrc_l, adst_l, src_my, dst_my, src_ot, dst_ot, exb, exb2,
             zvec, rb0, rb1, rb2, rb3, rb4, rb5, den_sh, out_sh,
             sg0, sg1, sg2, sg3, sg4, sg5,
             ss0, ss1, ss2, ss3, ss4, ss5, sem_a):
    c = lax.axis_index("c")
    t = lax.axis_index("s")
    w_my = c * NS + t
    w_ot = (1 - c) * NS + t
    base = t * RPT

    pltpu.async_copy(asrc_hbm, asrc_l, sg0)
    pltpu.async_copy(adst_hbm, adst_l, sg1)
    pltpu.async_copy(src_hbm.at[w_my], src_my, sg2)
    pltpu.async_copy(dst_hbm.at[w_my], dst_my, sg3)
    pltpu.async_copy(src_hbm.at[w_ot], src_ot, sg4)
    pltpu.async_copy(dst_hbm.at[w_ot], dst_ot, sg5)

    zv = jnp.zeros((16,), jnp.float32)

    def _z_row(r, _):
        for q in range(DP // 16):
            rb0[r, pl.ds(q * 16, 16)] = zv
        return 0
    lax.fori_loop(0, CH, _z_row, 0)

    def _z_vec(i, _):
        zvec[pl.ds(i * 16, 16)] = zv
        return 0
    lax.fori_loop(0, RPT // 16, _z_vec, 0)

    pltpu.sync_copy(zvec, den_sh.at[pl.ds(base, RPT)])
    pltpu.make_async_copy(asrc_hbm, asrc_l, sg0).wait()
    pltpu.make_async_copy(adst_hbm, adst_l, sg1).wait()
    pltpu.make_async_copy(src_hbm.at[w_my], src_my, sg2).wait()
    pltpu.make_async_copy(dst_hbm.at[w_my], dst_my, sg3).wait()
    pltpu.make_async_copy(src_hbm.at[w_ot], src_ot, sg4).wait()
    pltpu.make_async_copy(dst_hbm.at[w_ot], dst_ot, sg5).wait()
    plsc.subcore_barrier()


    for kp in range(NP // NC):
        p = 2 * kp + c

        lax.fori_loop(0, CH, _z_row, 0)
        for i in range(RPT // CH):
            pltpu.sync_copy(rb0, out_sh.at[pl.ds(base + i * CH, CH)])
        plsc.subcore_barrier()
        if kp == 1:
            pltpu.sync_copy(den_sh.at[pl.ds(base, RPT)],
                            den_hbm.at[c].at[pl.ds(base, RPT)])

        def _rows(sref, dref, eref, compute_ex):
            rbs = (rb0, rb1, rb2, rb3, rb4, rb5)
            sgs = (sg0, sg1, sg2, sg3, sg4, sg5)
            sss = (ss0, ss1, ss2, ss3, ss4, ss5)
            LA = 4

            def _start_g(j, b):
                pltpu.async_copy(hp_hbm.at[p].at[sref.at[j]], rbs[b], sgs[b])

            def _wait_g(b):
                pltpu.make_async_copy(hp_hbm.at[p].at[sref.at[0]], rbs[b],
                                      sgs[b]).wait()

            def _wait_s(b):
                pltpu.make_async_copy(rbs[b], out_sh.at[dref.at[0]],
                                      sss[b]).wait()

            def _scale(j, b):
                rb = rbs[b]

                def _sc16(g, _):
                    sl16 = pl.ds(g * 16, 16)
                    if compute_ex:
                        e = (plsc.load_gather(asrc_l, [sref[j, sl16]]) +
                             plsc.load_gather(adst_l, [dref[j, sl16]]))
                        e = jnp.where(e >= 0.0, e, e * 0.2)
                        avec = jnp.exp(e)
                        eref[j, sl16] = avec
                    else:
                        avec = eref[j, sl16]
                    for r in range(16):
                        a = avec[r]
                        row = g * 16 + r
                        for q in range(DP // 16):
                            sl = pl.ds(q * 16, 16)
                            rb[row, sl] = rb[row, sl] * a
                    return 0
                lax.fori_loop(0, CH // 16, _sc16, 0)

            for b in range(LA):
                _start_g(b, b)

            def _body(j2, _):
                for u in range(6):
                    j = 6 * j2 + u
                    bn = (u + LA) % 6

                    @pl.when(j < J)
                    def _():
                        _wait_g(u)
                        _scale(j, u)
                        pltpu.async_copy(rbs[u], out_sh.at[dref.at[j]],
                                         sss[u], add=True)
                        if compute_ex:
                            pltpu.async_copy(eref.at[j],
                                             den_sh.at[dref.at[j]],
                                             sem_a, add=True)

                        @pl.when(j >= 2)
                        def _():
                            _wait_s(bn)

                        @pl.when(j + LA < J)
                        def _():
                            _start_g(j + LA, bn)
                return 0
            lax.fori_loop(0, (J + 5) // 6, _body, 0)
            _wait_s((J - 2) % 6)
            _wait_s((J - 1) % 6)

        _rows(src_my, dst_my, exb, kp == 0)
        _rows(src_ot, dst_ot, exb2, kp == 0)
        if kp == 0:
            def _drain_a(i, _):
                pltpu.make_async_copy(exb.at[0], den_sh.at[dst_my.at[0]],
                                      sem_a).wait()
                return 0
            lax.fori_loop(0, 2 * J, _drain_a, 0)
        plsc.subcore_barrier()

        pltpu.sync_copy(out_sh.at[pl.ds(base, RPT)],
                        out_hbm.at[pl.ds(base, RPT), pl.ds(p * DP, DP)])
        plsc.subcore_barrier()


def kernel(x, edge_index, batch_size, framework,
           W1, a_src1, a_dst1, b1,
           W2, a_src2, a_dst2, b2,
           W3, a_src3, a_dst3, b3):
    f32 = jnp.float32
    loop = jnp.arange(N, dtype=jnp.int32)
    src = jnp.concatenate(
        [edge_index[0], loop, jnp.zeros((PAD,), jnp.int32)]).reshape(NW, J, CH)
    dst = jnp.concatenate(
        [edge_index[1], loop, jnp.full((PAD,), N, jnp.int32)]).reshape(NW, J, CH)

    def edge_phase(h4, s, d):
        agg, den = _sc_edge(h4, s.reshape(NN), d.reshape(NN), src, dst)
        return agg, den[0].reshape(NN, 1)

    h4, s, d = _mm1(x, W1, a_src1.reshape(1, D), a_dst1.reshape(1, D))
    agg, den = edge_phase(h4, s, d)
    h4, s, d = _mm2(agg, den, b1.reshape(1, D), W2,
                    a_src2.reshape(1, D), a_dst2.reshape(1, D))
    agg, den = edge_phase(h4, s, d)
    h4, s, d = _mm2(agg, den, b2.reshape(1, D), W3,
                    a_src3.reshape(1, D), a_dst3.reshape(1, D))
    agg, den = edge_phase(h4, s, d)

    limit = jnp.where(framework != 0, jnp.asarray(N, dtype=jnp.int32),
                      batch_size)
    keep = (jnp.arange(NN, dtype=jnp.int32) < limit).astype(f32).reshape(NN, 1)
    return _fin(agg, den, b3.reshape(1, D), keep)


# speedup vs baseline: 1.1155x; 1.0029x over previous
import functools

import jax
import jax.numpy as jnp
from jax import lax
from jax.experimental import pallas as pl
from jax.experimental.pallas import tpu as pltpu
from jax.experimental.pallas import tpu_sc as plsc

N = 10000
D = 128
E = 320000
NN = 10240
NC = 2
NS = 16
NW = NC * NS
CH = 128
J = 81
EPW = J * CH
ET = NW * EPW
ER = E + N
PAD = ET - ER
RPT = NN // NS
DP = 32
NP = D // DP


def _split_store(h, h4_ref):
    for p in range(NP):
        h4_ref[p, :, :] = h[:, p * DP:(p + 1) * DP]


def _mm1_body(x_ref, w_ref, asv_ref, adv_ref, h4_ref, s_ref, d_ref):
    x = x_ref[...]
    h = lax.dot_general(x, w_ref[...], (((1,), (1,)), ((), ())),
                        preferred_element_type=jnp.float32)
    h = jnp.concatenate([h, jnp.zeros((NN - N, D), jnp.float32)], axis=0)
    _split_store(h, h4_ref)
    s_ref[...] = jnp.sum(h * asv_ref[...], axis=1, keepdims=True)
    d_ref[...] = jnp.sum(h * adv_ref[...], axis=1, keepdims=True)


def _mm2_body(g_ref, den_ref, b_ref, w_ref, asv_ref, adv_ref,
              h4_ref, s_ref, d_ref):
    y = jnp.maximum(g_ref[...] / (den_ref[...] + 1e-16) + b_ref[...], 0.0)
    h = lax.dot_general(y, w_ref[...], (((1,), (1,)), ((), ())),
                        preferred_element_type=jnp.float32)
    rows = lax.broadcasted_iota(jnp.int32, (NN, 1), 0)
    h = jnp.where(rows < N, h, 0.0)
    _split_store(h, h4_ref)
    s_ref[...] = jnp.sum(h * asv_ref[...], axis=1, keepdims=True)
    d_ref[...] = jnp.sum(h * adv_ref[...], axis=1, keepdims=True)


def _fin_body(g_ref, den_ref, b_ref, keep_ref, o_ref):
    y = jnp.maximum(g_ref[...] / (den_ref[...] + 1e-16) + b_ref[...], 0.0)
    o_ref[...] = (y * keep_ref[...])[:N, :]


_MM_OUT = [jax.ShapeDtypeStruct((NP, NN, DP), jnp.float32),
           jax.ShapeDtypeStruct((NN, 1), jnp.float32),
           jax.ShapeDtypeStruct((NN, 1), jnp.float32)]


def _mm1(xp, w, asv, adv):
    return pl.pallas_call(_mm1_body, out_shape=_MM_OUT)(xp, w, asv, adv)


def _mm2(agg, den, b, w, asv, adv):
    return pl.pallas_call(_mm2_body, out_shape=_MM_OUT)(agg, den, b, w,
                                                        asv, adv)


def _fin(agg, den, b, keep):
    return pl.pallas_call(
        _fin_body, out_shape=jax.ShapeDtypeStruct((N, D), jnp.float32),
    )(agg, den, b, keep)


_sc_mesh = plsc.VectorSubcoreMesh(
    core_axis_name="c", subcore_axis_name="s", num_cores=NC, num_subcores=NS)


@functools.partial(
    pl.kernel,
    out_type=[jax.ShapeDtypeStruct((NN, D), jnp.float32),
              jax.ShapeDtypeStruct((NC, NN), jnp.float32)],
    mesh=_sc_mesh,
    compiler_params=pltpu.CompilerParams(needs_layout_passes=False,
                                         use_tc_tiling_on_sc=False),
    scratch_types=[
        pltpu.VMEM((NN,), jnp.float32),
        pltpu.VMEM((NN,), jnp.float32),
        pltpu.VMEM((J, CH), jnp.int32),
        pltpu.VMEM((J, CH), jnp.int32),
        pltpu.VMEM((J, CH), jnp.int32),
        pltpu.VMEM((J, CH), jnp.int32),
        pltpu.VMEM((J, CH), jnp.float32),
        pltpu.VMEM((J, CH), jnp.float32),
        pltpu.VMEM((RPT,), jnp.float32),
        pltpu.VMEM((CH, DP), jnp.float32),
        pltpu.VMEM((CH, DP), jnp.float32),
        pltpu.VMEM((CH, DP), jnp.float32),
        pltpu.VMEM((CH, DP), jnp.float32),
        pltpu.VMEM((CH, DP), jnp.float32),
        pltpu.VMEM((CH, DP), jnp.float32),
        pltpu.VMEM_SHARED((NN,), jnp.float32),
        pltpu.VMEM_SHARED((NN, DP), jnp.float32),
        pltpu.SemaphoreType.DMA,
        pltpu.SemaphoreType.DMA,
        pltpu.SemaphoreType.DMA,
        pltpu.SemaphoreType.DMA,
        pltpu.SemaphoreType.DMA,
        pltpu.SemaphoreType.DMA,
        pltpu.SemaphoreType.DMA,
        pltpu.SemaphoreType.DMA,
        pltpu.SemaphoreType.DMA,
        pltpu.SemaphoreType.DMA,
        pltpu.SemaphoreType.DMA,
        pltpu.SemaphoreType.DMA,
        pltpu.SemaphoreType.DMA,
    ],
)
def _sc_edge(hp_hbm, asrc_hbm, adst_hbm, src_hbm, dst_hbm,
             out_hbm, den_hbm,
             asrc_l, adst_l, src_my, dst_my, src_ot, dst_ot, exb, exb2,
             zvec, rb0, rb1, rb2, rb3, rb4, rb5, den_sh, out_sh,
             sg0, sg1, sg2, sg3, sg4, sg5,
             ss0, ss1, ss2, ss3, ss4, ss5, sem_a):
    c = lax.axis_index("c")
    t = lax.axis_index("s")
    w_my = c * NS + t
    w_ot = (1 - c) * NS + t
    base = t * RPT

    pltpu.async_copy(asrc_hbm, asrc_l, sg0)
    pltpu.async_copy(adst_hbm, adst_l, sg1)
    pltpu.async_copy(src_hbm.at[w_my], src_my, sg2)
    pltpu.async_copy(dst_hbm.at[w_my], dst_my, sg3)
    pltpu.async_copy(src_hbm.at[w_ot], src_ot, sg4)
    pltpu.async_copy(dst_hbm.at[w_ot], dst_ot, sg5)

    zv = jnp.zeros((16,), jnp.float32)

    def _z_row(r, _):
        for q in range(DP // 16):
            rb0[r, pl.ds(q * 16, 16)] = zv
        return 0
    lax.fori_loop(0, CH, _z_row, 0)

    def _z_vec(i, _):
        zvec[pl.ds(i * 16, 16)] = zv
        return 0
    lax.fori_loop(0, RPT // 16, _z_vec, 0)

    pltpu.sync_copy(zvec, den_sh.at[pl.ds(base, RPT)])
    pltpu.make_async_copy(asrc_hbm, asrc_l, sg0).wait()
    pltpu.make_async_copy(adst_hbm, adst_l, sg1).wait()
    pltpu.make_async_copy(src_hbm.at[w_my], src_my, sg2).wait()
    pltpu.make_async_copy(dst_hbm.at[w_my], dst_my, sg3).wait()
    pltpu.make_async_copy(src_hbm.at[w_ot], src_ot, sg4).wait()
    pltpu.make_async_copy(dst_hbm.at[w_ot], dst_ot, sg5).wait()
    plsc.subcore_barrier()


    for kp in range(NP // NC):
        p = 2 * kp + c

        lax.fori_loop(0, CH, _z_row, 0)
        zsems = (sg0, sg1, sg2, sg3, sg4)
        for i in range(RPT // CH):
            pltpu.async_copy(rb0, out_sh.at[pl.ds(base + i * CH, CH)],
                             zsems[i])
        for i in range(RPT // CH):
            pltpu.make_async_copy(rb0, out_sh.at[pl.ds(base + i * CH, CH)],
                                  zsems[i]).wait()
        plsc.subcore_barrier()
        if kp == 1:
            pltpu.sync_copy(den_sh.at[pl.ds(base, RPT)],
                            den_hbm.at[c].at[pl.ds(base, RPT)])

        def _rows(sref, dref, eref, compute_ex):
            rbs = (rb0, rb1, rb2, rb3, rb4, rb5)
            sgs = (sg0, sg1, sg2, sg3, sg4, sg5)
            sss = (ss0, ss1, ss2, ss3, ss4, ss5)
            LA = 4

            def _start_g(j, b):
                pltpu.async_copy(hp_hbm.at[p].at[sref.at[j]], rbs[b], sgs[b])

            def _wait_g(b):
                pltpu.make_async_copy(hp_hbm.at[p].at[sref.at[0]], rbs[b],
                                      sgs[b]).wait()

            def _wait_s(b):
                pltpu.make_async_copy(rbs[b], out_sh.at[dref.at[0]],
                                      sss[b]).wait()

            def _scale(j, b):
                rb = rbs[b]

                def _sc16(g, _):
                    sl16 = pl.ds(g * 16, 16)
                    if compute_ex:
                        e = (plsc.load_gather(asrc_l, [sref[j, sl16]]) +
                             plsc.load_gather(adst_l, [dref[j, sl16]]))
                        e = jnp.where(e >= 0.0, e, e * 0.2)
                        avec = jnp.exp(e)
                        eref[j, sl16] = avec
                    else:
                        avec = eref[j, sl16]
                    for r in range(16):
                        a = avec[r]
                        row = g * 16 + r
                        for q in range(DP // 16):
                            sl = pl.ds(q * 16, 16)
                            rb[row, sl] = rb[row, sl] * a
                    return 0
                lax.fori_loop(0, CH // 16, _sc16, 0)

            for b in range(LA):
                _start_g(b, b)

            def _body(j2, _):
                for u in range(6):
                    j = 6 * j2 + u
                    bn = (u + LA) % 6

                    @pl.when(j < J)
                    def _():
                        _wait_g(u)
                        _scale(j, u)
                        pltpu.async_copy(rbs[u], out_sh.at[dref.at[j]],
                                         sss[u], add=True)
                        if compute_ex:
                            pltpu.async_copy(eref.at[j],
                                             den_sh.at[dref.at[j]],
                                             sem_a, add=True)

                        @pl.when(j >= 2)
                        def _():
                            _wait_s(bn)

                        @pl.when(j + LA < J)
                        def _():
                            _start_g(j + LA, bn)
                return 0
            lax.fori_loop(0, (J + 5) // 6, _body, 0)
            _wait_s((J - 2) % 6)
            _wait_s((J - 1) % 6)

        _rows(src_my, dst_my, exb, kp == 0)
        _rows(src_ot, dst_ot, exb2, kp == 0)
        if kp == 0:
            def _drain_a(i, _):
                pltpu.make_async_copy(exb.at[0], den_sh.at[dst_my.at[0]],
                                      sem_a).wait()
                return 0
            lax.fori_loop(0, 2 * J, _drain_a, 0)
        plsc.subcore_barrier()

        pltpu.sync_copy(out_sh.at[pl.ds(base, RPT)],
                        out_hbm.at[pl.ds(base, RPT), pl.ds(p * DP, DP)])
        plsc.subcore_barrier()


def kernel(x, edge_index, batch_size, framework,
           W1, a_src1, a_dst1, b1,
           W2, a_src2, a_dst2, b2,
           W3, a_src3, a_dst3, b3):
    f32 = jnp.float32
    loop = jnp.arange(N, dtype=jnp.int32)
    src = jnp.concatenate(
        [edge_index[0], loop, jnp.zeros((PAD,), jnp.int32)]).reshape(NW, J, CH)
    dst = jnp.concatenate(
        [edge_index[1], loop, jnp.full((PAD,), N, jnp.int32)]).reshape(NW, J, CH)

    def edge_phase(h4, s, d):
        agg, den = _sc_edge(h4, s.reshape(NN), d.reshape(NN), src, dst)
        return agg, den[0].reshape(NN, 1)

    h4, s, d = _mm1(x, W1, a_src1.reshape(1, D), a_dst1.reshape(1, D))
    agg, den = edge_phase(h4, s, d)
    h4, s, d = _mm2(agg, den, b1.reshape(1, D), W2,
                    a_src2.reshape(1, D), a_dst2.reshape(1, D))
    agg, den = edge_phase(h4, s, d)
    h4, s, d = _mm2(agg, den, b2.reshape(1, D), W3,
                    a_src3.reshape(1, D), a_dst3.reshape(1, D))
    agg, den = edge_phase(h4, s, d)

    limit = jnp.where(framework != 0, jnp.asarray(N, dtype=jnp.int32),
                      batch_size)
    keep = (jnp.arange(NN, dtype=jnp.int32) < limit).astype(f32).reshape(NN, 1)
    return _fin(agg, den, b3.reshape(1, D), keep)
